# baseline (speedup 1.0000x reference)
import functools

import jax
import jax.numpy as jnp
from jax import lax
from jax.experimental import pallas as pl
from jax.experimental.pallas import tpu as pltpu
from jax.experimental.pallas import tpu_sc as plsc

D = 64
DP = 128
NC = 2
NS = 16
LANES = 16
CHUNK = 64
NBUF = 4
NOBUF = 2
V_BLK = 4096


@functools.lru_cache(maxsize=None)
def _build_transpose(V):
    grid = pl.cdiv(V, V_BLK)

    def body(in_ref, o_ref):
        o_ref[:, :D] = in_ref[...].T

    return pl.pallas_call(
        body,
        grid=(grid,),
        in_specs=[pl.BlockSpec((D, V_BLK), lambda i: (0, i))],
        out_specs=pl.BlockSpec((V_BLK, DP), lambda i: (i, 0)),
        out_shape=jax.ShapeDtypeStruct((V, DP), jnp.float32),
    )


@functools.lru_cache(maxsize=None)
def _build_lookup(N, V, L_POS):
    NW = NC * NS
    per_w = N // NW
    nch = per_w // CHUNK

    mesh = plsc.VectorSubcoreMesh(core_axis_name="c", subcore_axis_name="s")

    @functools.partial(
        pl.kernel,
        mesh=mesh,
        out_type=jax.ShapeDtypeStruct((N // 2, DP), jnp.float32),
        compiler_params=pltpu.CompilerParams(
            use_tc_tiling_on_sc=True, needs_layout_passes=False),
        scratch_types=[pltpu.VMEM((per_w,), jnp.int32),
                       pltpu.VMEM((L_POS // 2, DP), jnp.float32)]
                      + [pltpu.VMEM((CHUNK, DP), jnp.float32) for _ in range(NBUF)]
                      + [pltpu.VMEM((CHUNK // 2, DP), jnp.float32) for _ in range(NOBUF)]
                      + [pltpu.SemaphoreType.DMA for _ in range(NBUF)]
                      + [pltpu.SemaphoreType.DMA for _ in range(NOBUF)],
    )
    def k(x_hbm, tok_hbm, pos_hbm, out_hbm, idx_v, pos_v, *rest):
        o = 0
        gbufs = rest[o:o + NBUF]; o += NBUF
        obufs = rest[o:o + NOBUF]; o += NOBUF
        gsems = rest[o:o + NBUF]; o += NBUF
        osems = rest[o:o + NOBUF]
        wid = lax.axis_index("s") * NC + lax.axis_index("c")
        base = wid * per_w

        pltpu.sync_copy(x_hbm.at[pl.ds(base, per_w)], idx_v)
        pltpu.sync_copy(pos_hbm, pos_v)

        def gather_start(c, b):
            pltpu.async_copy(
                tok_hbm.at[idx_v.at[pl.ds(c * CHUNK, CHUNK)]], gbufs[b], gsems[b])

        def gather_wait(b):
            pltpu.make_async_copy(
                tok_hbm.at[pl.ds(0, CHUNK)], gbufs[b], gsems[b]).wait()

        def compute(c, b, ob):
            pbase = (c * CHUNK) % L_POS

            def row(r, carry):
                ph = (pbase + r) // 2
                po = (r % 2) * D
                q = r // 2
                for j in range(D // LANES):
                    v = gbufs[b][r, pl.ds(j * LANES, LANES)]
                    p = pos_v[ph, pl.ds(po + j * LANES, LANES)]
                    obufs[ob][q, pl.ds(po + j * LANES, LANES)] = v + p
                return carry
            lax.fori_loop(0, CHUNK, row, 0)

        def out_start(c, ob):
            off = pl.multiple_of(base // 2 + c * (CHUNK // 2), CHUNK // 2)
            pltpu.async_copy(
                obufs[ob],
                out_hbm.at[pl.ds(off, CHUNK // 2)],
                osems[ob])

        def out_wait(ob):
            pltpu.make_async_copy(
                out_hbm.at[pl.ds(0, CHUNK // 2)], obufs[ob], osems[ob]).wait()

        def do_chunk(c, b, ob, wait_prev_out, start_prev_next):
            gather_wait(b)
            compute(c, b, ob)
            out_start(c, ob)
            if wait_prev_out:
                out_wait(1 - ob)
            if start_prev_next:
                gather_start(c - 1 + NBUF, (b - 1) % NBUF)

        for b in range(NBUF):
            gather_start(b, b)

        for b in range(NBUF):
            do_chunk(b, b, b % NOBUF, b >= 1, b >= 1)

        def group(g, carry):
            c0 = g * NBUF
            for b in range(NBUF):
                do_chunk(c0 + b, b, b % NOBUF, True, True)
            return carry
        lax.fori_loop(1, nch // NBUF - 1, group, 0)
        for b in range(NBUF):
            c = nch - NBUF + b
            do_chunk(c, b, b % NOBUF, True, c - 1 + NBUF < nch)
        out_wait((nch - 1) % NOBUF)

    return k


def kernel(x, token_table, pos_table):
    B, L = x.shape
    V = token_table.shape[0]
    xf = x.reshape(B * L).astype(jnp.int32)
    tok_p = _build_transpose(V)(token_table.T)
    pos_q = pos_table.reshape(L // 2, 2 * D)
    out = _build_lookup(B * L, V, L)(xf, tok_p, pos_q)
    return out.reshape(B, L, D)


# speedup vs baseline: 1.5732x; 1.5732x over previous
import functools

import jax
import jax.numpy as jnp
from jax import lax
from jax.experimental import pallas as pl
from jax.experimental.pallas import tpu as pltpu
from jax.experimental.pallas import tpu_sc as plsc

D = 64
DP = 128
NC = 2
NS = 16
LANES = 16
CHUNK = 128
NBUF = 4
V_BLK = 8192


@functools.lru_cache(maxsize=None)
def _build_transpose(V):
    grid = pl.cdiv(V, V_BLK)

    def body(in_ref, o_ref):
        o_ref[:, :D] = in_ref[...].T

    return pl.pallas_call(
        body,
        grid=(grid,),
        in_specs=[pl.BlockSpec((D, V_BLK), lambda i: (0, i))],
        out_specs=pl.BlockSpec((V_BLK, DP), lambda i: (i, 0)),
        out_shape=jax.ShapeDtypeStruct((V, DP), jnp.float32),
    )


@functools.lru_cache(maxsize=None)
def _build_lookup(N, V, L_POS):
    NW = NC * NS
    per_w = N // NW
    nch = per_w // CHUNK

    mesh = plsc.VectorSubcoreMesh(core_axis_name="c", subcore_axis_name="s")

    @functools.partial(
        pl.kernel,
        mesh=mesh,
        out_type=jax.ShapeDtypeStruct((N, DP), jnp.float32),
        compiler_params=pltpu.CompilerParams(
            use_tc_tiling_on_sc=True, needs_layout_passes=False),
        scratch_types=[pltpu.VMEM((per_w,), jnp.int32),
                       pltpu.VMEM((L_POS // 2, DP), jnp.float32)]
                      + [pltpu.VMEM((CHUNK, DP), jnp.float32) for _ in range(NBUF)]
                      + [pltpu.SemaphoreType.DMA for _ in range(NBUF)]
                      + [pltpu.SemaphoreType.DMA for _ in range(NBUF)],
    )
    def k(x_hbm, tok_hbm, pos_hbm, out_hbm, idx_v, pos_v, *rest):
        bufs = rest[:NBUF]
        sems = rest[NBUF:NBUF + NBUF]
        osems = rest[NBUF + NBUF:]
        wid = lax.axis_index("s") * NC + lax.axis_index("c")
        base = wid * per_w

        pltpu.sync_copy(x_hbm.at[pl.ds(base, per_w)], idx_v)
        pltpu.sync_copy(pos_hbm, pos_v)

        def gather_start(c, b):
            pltpu.async_copy(
                tok_hbm.at[idx_v.at[pl.ds(c * CHUNK, CHUNK)]], bufs[b], sems[b])

        def gather_wait(b):
            pltpu.make_async_copy(
                tok_hbm.at[pl.ds(0, CHUNK)], bufs[b], sems[b]).wait()

        def add_pos(b, c):
            pbase = (c * CHUNK) % L_POS

            def row(r, carry):
                pr = pbase + r
                ph = pr // 2
                po = (pr % 2) * D
                for j in range(D // LANES):
                    plsc.addupdate(bufs[b].at[r, pl.ds(j * LANES, LANES)],
                                   pos_v[ph, pl.ds(po + j * LANES, LANES)])
                return carry
            lax.fori_loop(0, CHUNK, row, 0)

        def out_start(c, b):
            pltpu.async_copy(bufs[b],
                             out_hbm.at[pl.ds(base + c * CHUNK, CHUNK)],
                             osems[b])

        def out_wait(b):
            pltpu.make_async_copy(
                out_hbm.at[pl.ds(0, CHUNK)], bufs[b], osems[b]).wait()

        def do_chunk(c, b, handle_prev, start_prev_next):
            gather_wait(b)
            add_pos(b, c)
            out_start(c, b)
            if handle_prev:
                pb = (b - 1) % NBUF
                out_wait(pb)
                if start_prev_next:
                    gather_start(c - 1 + NBUF, pb)

        for b in range(NBUF):
            gather_start(b, b)

        for b in range(NBUF):
            do_chunk(b, b, b >= 1, True)

        def group(g, carry):
            c0 = g * NBUF
            for b in range(NBUF):
                do_chunk(c0 + b, b, True, True)
            return carry
        lax.fori_loop(1, nch // NBUF - 1, group, 0)
        for b in range(NBUF):
            c = nch - NBUF + b
            do_chunk(c, b, True, c - 1 + NBUF < nch)
        out_wait((nch - 1) % NBUF)

    return k


def kernel(x, token_table, pos_table):
    B, L = x.shape
    V = token_table.shape[0]
    xf = x.reshape(B * L).astype(jnp.int32)
    tok_p = _build_transpose(V)(token_table.T)
    pos_q = pos_table.reshape(L // 2, 2 * D)
    out = _build_lookup(B * L, V, L)(xf, tok_p, pos_q)
    return out[:, :D].reshape(B, L, D)


# speedup vs baseline: 1.6202x; 1.0299x over previous
import functools

import jax
import jax.numpy as jnp
from jax import lax
from jax.experimental import pallas as pl
from jax.experimental.pallas import tpu as pltpu
from jax.experimental.pallas import tpu_sc as plsc

D = 64
DP = 128
NC = 2
NS = 16
LANES = 16
CHUNK = 128
NBUF = 4
V_BLK = 16384


@functools.lru_cache(maxsize=None)
def _build_transpose(V):
    grid = pl.cdiv(V, V_BLK)

    def body(in_ref, o_ref):
        o_ref[:, :D] = in_ref[...].T

    return pl.pallas_call(
        body,
        grid=(grid,),
        in_specs=[pl.BlockSpec((D, V_BLK), lambda i: (0, i))],
        out_specs=pl.BlockSpec((V_BLK, DP), lambda i: (i, 0)),
        out_shape=jax.ShapeDtypeStruct((V, DP), jnp.float32),
    )


@functools.lru_cache(maxsize=None)
def _build_lookup(N, V, L_POS):
    NW = NC * NS
    per_w = N // NW
    nch = per_w // CHUNK

    mesh = plsc.VectorSubcoreMesh(core_axis_name="c", subcore_axis_name="s")

    @functools.partial(
        pl.kernel,
        mesh=mesh,
        out_type=jax.ShapeDtypeStruct((N, DP), jnp.float32),
        compiler_params=pltpu.CompilerParams(
            use_tc_tiling_on_sc=True, needs_layout_passes=False),
        scratch_types=[pltpu.VMEM((per_w,), jnp.int32),
                       pltpu.VMEM((L_POS // 2, DP), jnp.float32)]
                      + [pltpu.VMEM((CHUNK, DP), jnp.float32) for _ in range(NBUF)]
                      + [pltpu.SemaphoreType.DMA for _ in range(NBUF)]
                      + [pltpu.SemaphoreType.DMA for _ in range(NBUF)],
    )
    def k(x_hbm, tok_hbm, pos_hbm, out_hbm, idx_v, pos_v, *rest):
        bufs = rest[:NBUF]
        sems = rest[NBUF:NBUF + NBUF]
        osems = rest[NBUF + NBUF:]
        wid = lax.axis_index("s") * NC + lax.axis_index("c")
        base = wid * per_w

        pltpu.sync_copy(x_hbm.at[pl.ds(base, per_w)], idx_v)
        pltpu.sync_copy(pos_hbm, pos_v)

        def gather_start(c, b):
            pltpu.async_copy(
                tok_hbm.at[idx_v.at[pl.ds(c * CHUNK, CHUNK)]], bufs[b], sems[b])

        def gather_wait(b):
            pltpu.make_async_copy(
                tok_hbm.at[pl.ds(0, CHUNK)], bufs[b], sems[b]).wait()

        def add_pos(b, c):
            pbase = (c * CHUNK) % L_POS

            def row(r, carry):
                pr = pbase + r
                ph = pr // 2
                po = (pr % 2) * D
                for j in range(D // LANES):
                    plsc.addupdate(bufs[b].at[r, pl.ds(j * LANES, LANES)],
                                   pos_v[ph, pl.ds(po + j * LANES, LANES)])
                return carry
            lax.fori_loop(0, CHUNK, row, 0)

        def out_start(c, b):
            pltpu.async_copy(bufs[b],
                             out_hbm.at[pl.ds(base + c * CHUNK, CHUNK)],
                             osems[b])

        def out_wait(b):
            pltpu.make_async_copy(
                out_hbm.at[pl.ds(0, CHUNK)], bufs[b], osems[b]).wait()

        def do_chunk(c, b, handle_prev, start_prev_next):
            gather_wait(b)
            add_pos(b, c)
            out_start(c, b)
            if handle_prev:
                pb = (b - 1) % NBUF
                out_wait(pb)
                if start_prev_next:
                    gather_start(c - 1 + NBUF, pb)

        for b in range(NBUF):
            gather_start(b, b)

        for b in range(NBUF):
            do_chunk(b, b, b >= 1, True)

        def group(g, carry):
            c0 = g * NBUF
            for b in range(NBUF):
                do_chunk(c0 + b, b, True, True)
            return carry
        lax.fori_loop(1, nch // NBUF - 1, group, 0)
        for b in range(NBUF):
            c = nch - NBUF + b
            do_chunk(c, b, True, c - 1 + NBUF < nch)
        out_wait((nch - 1) % NBUF)

    return k


def kernel(x, token_table, pos_table):
    B, L = x.shape
    V = token_table.shape[0]
    xf = x.reshape(B * L).astype(jnp.int32)
    tok_p = _build_transpose(V)(token_table.T)
    pos_q = pos_table.reshape(L // 2, 2 * D)
    out = _build_lookup(B * L, V, L)(xf, tok_p, pos_q)
    return out[:, :D].reshape(B, L, D)
